# baseline (speedup 1.0000x reference)
import jax
import jax.numpy as jnp
from jax import lax
from jax.experimental import pallas as pl
from jax.experimental.pallas import tpu as pltpu
from jax.experimental.pallas import tpu_sc as plsc

N = 50000
D = 256
G = 64
NEG_INF = float("-inf")

TCCHUNK = 128
NCH = 278
TCROWS = TCCHUNK * NCH
SCROWS = N - TCROWS

L = 16
CB = D // L
NW = 32
CHUNK = 112
RPW = 448
NPAIRS = RPW // CHUNK // 2
LAST_W = NW - 1
TAIL_ROWS = SCROWS - NW * RPW
TAIL_GROUPS = TAIL_ROWS // L

IDS_PAD = ((N + TCCHUNK - 1) // TCCHUNK) * TCCHUNK


def _tree_reduce(xs, op):
    while len(xs) > 1:
        xs = [op(xs[2 * i], xs[2 * i + 1]) for i in range(len(xs) // 2)] + \
             (xs[-1:] if len(xs) % 2 else [])
    return xs[0]


def _sc_partials_kernel(x_hbm, ids_hbm, psum_hbm, pmax_hbm, pcnt_hbm,
                        x0, x1, i0, i1, sum_v, max_v, cnt_v,
                        sx0, sx1, si0, si1):
    wid = lax.axis_index("s") * 2 + lax.axis_index("c")
    base = TCROWS + wid * RPW

    zeros16 = jnp.zeros((L,), jnp.float32)
    neg16 = jnp.full((L,), NEG_INF, jnp.float32)
    ones16 = jnp.ones((L,), jnp.float32)

    def init_body(s, _):
        for cb in range(CB):
            sum_v[s, pl.ds(cb * L, L)] = zeros16
            max_v[s, pl.ds(cb * L, L)] = neg16
        cnt_v[s, :] = zeros16
        return 0
    lax.fori_loop(0, G, init_body, 0)

    xb = (x0, x1)
    ib = (i0, i1)
    sxb = (sx0, sx1)
    sib = (si0, si1)

    def start(c, k):
        st = base + c * CHUNK
        pltpu.async_copy(x_hbm.at[pl.ds(st, CHUNK)], xb[k], sxb[k])
        pltpu.async_copy(ids_hbm.at[pl.ds(st, CHUNK)],
                         ib[k].at[pl.ds(0, CHUNK)], sib[k])

    def wait(k):
        pltpu.make_async_copy(x_hbm.at[pl.ds(0, CHUNK)], xb[k], sxb[k]).wait()
        pltpu.make_async_copy(ids_hbm.at[pl.ds(0, CHUNK)],
                              ib[k].at[pl.ds(0, CHUNK)], sib[k]).wait()

    def process(x_v, ids_v, ngroups):
        def group_body(g, _):
            row0 = g * L
            bvec = ids_v[pl.ds(row0, L)]
            b0 = bvec[0]
            uniform = b0 == bvec[L - 1]

            def uniform_path():
                for cb in range(CB):
                    xs = [x_v[row0 + j, pl.ds(cb * L, L)] for j in range(L)]
                    s = _tree_reduce(list(xs), jnp.add)
                    m = _tree_reduce(list(xs), jnp.maximum)
                    plsc.addupdate(sum_v.at[b0, pl.ds(cb * L, L)], s)
                    cur = max_v[b0, pl.ds(cb * L, L)]
                    max_v[b0, pl.ds(cb * L, L)] = jnp.maximum(cur, m)
                plsc.addupdate(cnt_v.at[b0],
                               jnp.full((L,), float(L), jnp.float32))

            def rowwise_path():
                def row_body(j, _):
                    row = row0 + j
                    b = ids_v[pl.ds(row, L)][0]
                    for cb in range(CB):
                        x = x_v[row, pl.ds(cb * L, L)]
                        plsc.addupdate(sum_v.at[b, pl.ds(cb * L, L)], x)
                        cur = max_v[b, pl.ds(cb * L, L)]
                        max_v[b, pl.ds(cb * L, L)] = jnp.maximum(cur, x)
                    plsc.addupdate(cnt_v.at[b], ones16)
                    return 0
                lax.fori_loop(0, L, row_body, 0)

            lax.cond(uniform, uniform_path, rowwise_path)
            return 0
        lax.fori_loop(0, ngroups, group_body, 0)

    start(0, 0)

    def pair_body(p, _):
        c0 = 2 * p
        start(c0 + 1, 1)
        wait(0)
        process(x0, i0, CHUNK // L)

        @pl.when(p + 1 < NPAIRS)
        def _():
            start(c0 + 2, 0)
        wait(1)
        process(x1, i1, CHUNK // L)
        return 0
    lax.fori_loop(0, NPAIRS, pair_body, 0)

    @pl.when(wid == LAST_W)
    def _():
        st = base + NPAIRS * 2 * CHUNK
        pltpu.sync_copy(x_hbm.at[pl.ds(st, TAIL_ROWS)],
                        x0.at[pl.ds(0, TAIL_ROWS)])
        pltpu.sync_copy(ids_hbm.at[pl.ds(st, TAIL_ROWS)],
                        i0.at[pl.ds(0, TAIL_ROWS)])
        process(x0, i0, TAIL_GROUPS)

    pltpu.sync_copy(sum_v, psum_hbm.at[wid])
    pltpu.sync_copy(max_v, pmax_hbm.at[wid])
    pltpu.sync_copy(cnt_v, pcnt_hbm.at[wid])


def _tc_jagged_kernel(ids_ref, x_ref, sum_ref, max_ref, cnt_ref):
    i = pl.program_id(0)

    @pl.when(i == 0)
    def _():
        sum_ref[...] = jnp.zeros_like(sum_ref)
        max_ref[...] = jnp.full_like(max_ref, NEG_INF)
        cnt_ref[...] = jnp.zeros_like(cnt_ref)

    ids2 = ids_ref[0]
    bmin = jnp.min(ids2)
    bmax = jnp.max(ids2)
    seg = lax.broadcasted_iota(jnp.int32, (G, 1), 0)

    def accum(b, s, m, nrows):
        hit = seg == b
        oh = hit.astype(jnp.float32)
        sum_ref[...] += oh * s
        mb = jnp.where(hit, jnp.broadcast_to(m, (G, D)), NEG_INF)
        max_ref[...] = jnp.maximum(max_ref[...], mb)
        cnt_ref[...] += oh * nrows

    @pl.when(bmin == bmax)
    def _():
        x = x_ref[...]
        s = jnp.sum(x, axis=0, keepdims=True)
        m = jnp.max(x, axis=0, keepdims=True)
        accum(bmin, s, m, float(TCCHUNK))

    @pl.when(bmin != bmax)
    def _():
        riota = lax.broadcasted_iota(jnp.int32, (1, TCCHUNK), 1)

        def row_body(r, _):
            b = jnp.max(jnp.where(riota == r, ids2, 0))
            rowv = x_ref[pl.ds(r, 1), :]
            accum(b, rowv, rowv, 1.0)
            return 0
        lax.fori_loop(0, TCCHUNK, row_body, 0)


def _combine_kernel(tsum_ref, tmax_ref, tcnt_ref,
                    psum_ref, pmax_ref, pcnt_ref, w_ref, b_ref, out_ref):
    sums = tsum_ref[...] + jnp.sum(psum_ref[...], axis=0)
    maxs = jnp.maximum(tmax_ref[...], jnp.max(pmax_ref[...], axis=0))
    cnts = tcnt_ref[:, 0:1] + jnp.sum(pcnt_ref[...], axis=0)[:, 0:1]
    mean = sums / jnp.maximum(cnts, 1.0)
    maxs = jnp.where(maxs == NEG_INF, 0.0, maxs)
    combined = jnp.concatenate([mean, maxs], axis=1)
    proj = lax.dot_general(combined, w_ref[...],
                           (((1,), (1,)), ((), ())),
                           preferred_element_type=jnp.float32)
    out_ref[...] = proj + b_ref[...]


def kernel(node_embeddings, batch, W, b):
    batch = batch.astype(jnp.int32)
    ids_pad = jnp.pad(batch, (0, IDS_PAD - N)).reshape(-1, 1, TCCHUNK)

    sc = pl.kernel(
        _sc_partials_kernel,
        mesh=plsc.VectorSubcoreMesh(core_axis_name="c", subcore_axis_name="s"),
        out_type=[
            jax.ShapeDtypeStruct((NW, G, D), jnp.float32),
            jax.ShapeDtypeStruct((NW, G, D), jnp.float32),
            jax.ShapeDtypeStruct((NW, G, L), jnp.float32),
        ],
        scratch_types=[
            pltpu.VMEM((CHUNK, D), jnp.float32),
            pltpu.VMEM((CHUNK, D), jnp.float32),
            pltpu.VMEM((CHUNK + L,), jnp.int32),
            pltpu.VMEM((CHUNK + L,), jnp.int32),
            pltpu.VMEM((G, D), jnp.float32),
            pltpu.VMEM((G, D), jnp.float32),
            pltpu.VMEM((G, L), jnp.float32),
            pltpu.SemaphoreType.DMA,
            pltpu.SemaphoreType.DMA,
            pltpu.SemaphoreType.DMA,
            pltpu.SemaphoreType.DMA,
        ],
    )
    psum, pmax, pcnt = sc(node_embeddings, batch)

    tsum, tmax, tcnt = pl.pallas_call(
        _tc_jagged_kernel,
        grid=(NCH,),
        in_specs=[
            pl.BlockSpec((1, 1, TCCHUNK), lambda i: (i, 0, 0)),
            pl.BlockSpec((TCCHUNK, D), lambda i: (i, 0)),
        ],
        out_specs=[
            pl.BlockSpec((G, D), lambda i: (0, 0)),
            pl.BlockSpec((G, D), lambda i: (0, 0)),
            pl.BlockSpec((G, 128), lambda i: (0, 0)),
        ],
        out_shape=[
            jax.ShapeDtypeStruct((G, D), jnp.float32),
            jax.ShapeDtypeStruct((G, D), jnp.float32),
            jax.ShapeDtypeStruct((G, 128), jnp.float32),
        ],
        compiler_params=pltpu.CompilerParams(
            dimension_semantics=("arbitrary",)),
    )(ids_pad, node_embeddings)

    out = pl.pallas_call(
        _combine_kernel,
        out_shape=jax.ShapeDtypeStruct((G, D), jnp.float32),
    )(tsum, tmax, tcnt, psum, pmax, pcnt, W, b.reshape(1, D))
    return out


# speedup vs baseline: 7.2758x; 7.2758x over previous
import jax
import jax.numpy as jnp
from jax import lax
from jax.experimental import pallas as pl
from jax.experimental.pallas import tpu as pltpu
from jax.experimental.pallas import tpu_sc as plsc

N = 50000
D = 256
G = 64
NEG_INF = float("-inf")

TCCHUNK = 128
NCH = 278
TCROWS = TCCHUNK * NCH
SCROWS = N - TCROWS

L = 16
CB = D // L
NW = 32
CHUNK = 112
RPW = 448
NPAIRS = RPW // CHUNK // 2
LAST_W = NW - 1
TAIL_ROWS = SCROWS - NW * RPW
TAIL_GROUPS = TAIL_ROWS // L

IDS_PAD = ((N + TCCHUNK - 1) // TCCHUNK) * TCCHUNK


def _tree_reduce(xs, op):
    while len(xs) > 1:
        xs = [op(xs[2 * i], xs[2 * i + 1]) for i in range(len(xs) // 2)] + \
             (xs[-1:] if len(xs) % 2 else [])
    return xs[0]


def _sc_partials_kernel(x_hbm, ids_hbm, psum_hbm, pmax_hbm, pcnt_hbm,
                        x0, x1, i0, i1, sum_v, max_v, cnt_v,
                        sx0, sx1, si0, si1):
    wid = lax.axis_index("s") * 2 + lax.axis_index("c")
    base = TCROWS + wid * RPW

    zeros16 = jnp.zeros((L,), jnp.float32)
    neg16 = jnp.full((L,), NEG_INF, jnp.float32)
    ones16 = jnp.ones((L,), jnp.float32)

    def init_body(s, _):
        for cb in range(CB):
            sum_v[s, pl.ds(cb * L, L)] = zeros16
            max_v[s, pl.ds(cb * L, L)] = neg16
        cnt_v[s, :] = zeros16
        return 0
    lax.fori_loop(0, G, init_body, 0)

    xb = (x0, x1)
    ib = (i0, i1)
    sxb = (sx0, sx1)
    sib = (si0, si1)

    def start(c, k):
        st = base + c * CHUNK
        pltpu.async_copy(x_hbm.at[pl.ds(st, CHUNK)], xb[k], sxb[k])
        pltpu.async_copy(ids_hbm.at[pl.ds(st, CHUNK)],
                         ib[k].at[pl.ds(0, CHUNK)], sib[k])

    def wait(k):
        pltpu.make_async_copy(x_hbm.at[pl.ds(0, CHUNK)], xb[k], sxb[k]).wait()
        pltpu.make_async_copy(ids_hbm.at[pl.ds(0, CHUNK)],
                              ib[k].at[pl.ds(0, CHUNK)], sib[k]).wait()

    def process(x_v, ids_v, ngroups):
        def group_body(g, _):
            row0 = g * L
            bvec = ids_v[pl.ds(row0, L)]
            b0 = bvec[0]
            uniform = b0 == bvec[L - 1]

            def uniform_path():
                for cb in range(CB):
                    xs = [x_v[row0 + j, pl.ds(cb * L, L)] for j in range(L)]
                    s = _tree_reduce(list(xs), jnp.add)
                    m = _tree_reduce(list(xs), jnp.maximum)
                    plsc.addupdate(sum_v.at[b0, pl.ds(cb * L, L)], s)
                    cur = max_v[b0, pl.ds(cb * L, L)]
                    max_v[b0, pl.ds(cb * L, L)] = jnp.maximum(cur, m)
                plsc.addupdate(cnt_v.at[b0],
                               jnp.full((L,), float(L), jnp.float32))

            def rowwise_path():
                def row_body(j, _):
                    row = row0 + j
                    b = ids_v[pl.ds(row, L)][0]
                    for cb in range(CB):
                        x = x_v[row, pl.ds(cb * L, L)]
                        plsc.addupdate(sum_v.at[b, pl.ds(cb * L, L)], x)
                        cur = max_v[b, pl.ds(cb * L, L)]
                        max_v[b, pl.ds(cb * L, L)] = jnp.maximum(cur, x)
                    plsc.addupdate(cnt_v.at[b], ones16)
                    return 0
                lax.fori_loop(0, L, row_body, 0)

            lax.cond(uniform, uniform_path, rowwise_path)
            return 0
        lax.fori_loop(0, ngroups, group_body, 0)

    start(0, 0)

    def pair_body(p, _):
        c0 = 2 * p
        start(c0 + 1, 1)
        wait(0)
        process(x0, i0, CHUNK // L)

        @pl.when(p + 1 < NPAIRS)
        def _():
            start(c0 + 2, 0)
        wait(1)
        process(x1, i1, CHUNK // L)
        return 0
    lax.fori_loop(0, NPAIRS, pair_body, 0)

    @pl.when(wid == LAST_W)
    def _():
        st = base + NPAIRS * 2 * CHUNK
        pltpu.sync_copy(x_hbm.at[pl.ds(st, TAIL_ROWS)],
                        x0.at[pl.ds(0, TAIL_ROWS)])
        pltpu.sync_copy(ids_hbm.at[pl.ds(st, TAIL_ROWS)],
                        i0.at[pl.ds(0, TAIL_ROWS)])
        process(x0, i0, TAIL_GROUPS)

    pltpu.sync_copy(sum_v, psum_hbm.at[wid])
    pltpu.sync_copy(max_v, pmax_hbm.at[wid])
    pltpu.sync_copy(cnt_v, pcnt_hbm.at[wid])


def _tc_jagged_kernel(cmin_ref, cmax_ref, ids_ref, x_ref,
                      sum_ref, max_ref, cnt_ref,
                      cur_sum, cur_max, cur_cnt, cur_seg):
    i = pl.program_id(0)
    seg = lax.broadcasted_iota(jnp.int32, (G, 1), 0)

    @pl.when(i == 0)
    def _():
        sum_ref[...] = jnp.zeros_like(sum_ref)
        max_ref[...] = jnp.full_like(max_ref, NEG_INF)
        cnt_ref[...] = jnp.zeros_like(cnt_ref)
        cur_sum[...] = jnp.zeros_like(cur_sum)
        cur_max[...] = jnp.full_like(cur_max, NEG_INF)
        cur_cnt[...] = jnp.zeros_like(cur_cnt)
        cur_seg[0] = cmin_ref[0]

    def flush():
        hit = seg == cur_seg[0]
        oh = hit.astype(jnp.float32)
        sum_ref[...] += oh * cur_sum[...]
        mb = jnp.where(hit, jnp.broadcast_to(cur_max[...], (G, D)), NEG_INF)
        max_ref[...] = jnp.maximum(max_ref[...], mb)
        cnt_ref[...] += oh * cur_cnt[...]

    def reset(new_seg):
        cur_sum[...] = jnp.zeros_like(cur_sum)
        cur_max[...] = jnp.full_like(cur_max, NEG_INF)
        cur_cnt[...] = jnp.zeros_like(cur_cnt)
        cur_seg[0] = new_seg

    lo = cmin_ref[i]
    hi = cmax_ref[i]

    @pl.when(lo == hi)
    def _():
        @pl.when(lo != cur_seg[0])
        def _():
            flush()
            reset(lo)
        x = x_ref[...]
        cur_sum[...] += jnp.sum(x, axis=0, keepdims=True)
        cur_max[...] = jnp.maximum(cur_max[...],
                                   jnp.max(x, axis=0, keepdims=True))
        cur_cnt[...] += float(TCCHUNK)

    @pl.when(lo != hi)
    def _():
        ids_col = ids_ref[...]
        x = x_ref[...]

        def seg_body(s, _):
            @pl.when(s != cur_seg[0])
            def _():
                flush()
                reset(s)
            rm = ids_col == s
            xs = jnp.where(rm, x, 0.0)
            xm = jnp.where(rm, x, NEG_INF)
            cur_sum[...] += jnp.sum(xs, axis=0, keepdims=True)
            cur_max[...] = jnp.maximum(cur_max[...],
                                       jnp.max(xm, axis=0, keepdims=True))
            nrows = jnp.sum(rm.astype(jnp.float32), axis=0, keepdims=True)
            cur_cnt[...] += jnp.broadcast_to(nrows, cur_cnt.shape)
            return 0
        lax.fori_loop(lo, hi + 1, seg_body, 0)

    @pl.when(i == NCH - 1)
    def _():
        flush()


def _combine_kernel(tsum_ref, tmax_ref, tcnt_ref,
                    psum_ref, pmax_ref, pcnt_ref, w_ref, b_ref, out_ref):
    sums = tsum_ref[...] + jnp.sum(psum_ref[...], axis=0)
    maxs = jnp.maximum(tmax_ref[...], jnp.max(pmax_ref[...], axis=0))
    cnts = tcnt_ref[:, 0:1] + jnp.sum(pcnt_ref[...], axis=0)[:, 0:1]
    mean = sums / jnp.maximum(cnts, 1.0)
    maxs = jnp.where(maxs == NEG_INF, 0.0, maxs)
    combined = jnp.concatenate([mean, maxs], axis=1)
    proj = lax.dot_general(combined, w_ref[...],
                           (((1,), (1,)), ((), ())),
                           preferred_element_type=jnp.float32)
    out_ref[...] = proj + b_ref[...]


def kernel(node_embeddings, batch, W, b):
    batch = batch.astype(jnp.int32)
    ids_pad = jnp.pad(batch, (0, IDS_PAD - N))

    sc = pl.kernel(
        _sc_partials_kernel,
        mesh=plsc.VectorSubcoreMesh(core_axis_name="c", subcore_axis_name="s"),
        out_type=[
            jax.ShapeDtypeStruct((NW, G, D), jnp.float32),
            jax.ShapeDtypeStruct((NW, G, D), jnp.float32),
            jax.ShapeDtypeStruct((NW, G, L), jnp.float32),
        ],
        scratch_types=[
            pltpu.VMEM((CHUNK, D), jnp.float32),
            pltpu.VMEM((CHUNK, D), jnp.float32),
            pltpu.VMEM((CHUNK + L,), jnp.int32),
            pltpu.VMEM((CHUNK + L,), jnp.int32),
            pltpu.VMEM((G, D), jnp.float32),
            pltpu.VMEM((G, D), jnp.float32),
            pltpu.VMEM((G, L), jnp.float32),
            pltpu.SemaphoreType.DMA,
            pltpu.SemaphoreType.DMA,
            pltpu.SemaphoreType.DMA,
            pltpu.SemaphoreType.DMA,
        ],
    )
    psum, pmax, pcnt = sc(node_embeddings, batch)

    ids_mat = ids_pad.reshape(-1, TCCHUNK)
    cmin = jnp.min(ids_mat[:NCH], axis=1)
    cmax = jnp.max(ids_mat[:NCH], axis=1)
    ids_col = ids_pad.reshape(-1, 1)

    tsum, tmax, tcnt = pl.pallas_call(
        _tc_jagged_kernel,
        grid_spec=pltpu.PrefetchScalarGridSpec(
            num_scalar_prefetch=2,
            grid=(NCH,),
            in_specs=[
                pl.BlockSpec((TCCHUNK, 1), lambda i, *_: (i, 0)),
                pl.BlockSpec((TCCHUNK, D), lambda i, *_: (i, 0)),
            ],
            out_specs=[
                pl.BlockSpec((G, D), lambda i, *_: (0, 0)),
                pl.BlockSpec((G, D), lambda i, *_: (0, 0)),
                pl.BlockSpec((G, 128), lambda i, *_: (0, 0)),
            ],
            scratch_shapes=[
                pltpu.VMEM((1, D), jnp.float32),
                pltpu.VMEM((1, D), jnp.float32),
                pltpu.VMEM((1, 128), jnp.float32),
                pltpu.SMEM((1,), jnp.int32),
            ],
        ),
        out_shape=[
            jax.ShapeDtypeStruct((G, D), jnp.float32),
            jax.ShapeDtypeStruct((G, D), jnp.float32),
            jax.ShapeDtypeStruct((G, 128), jnp.float32),
        ],
        compiler_params=pltpu.CompilerParams(
            dimension_semantics=("arbitrary",)),
    )(cmin, cmax, ids_col, node_embeddings)

    out = pl.pallas_call(
        _combine_kernel,
        out_shape=jax.ShapeDtypeStruct((G, D), jnp.float32),
    )(tsum, tmax, tcnt, psum, pmax, pcnt, W, b.reshape(1, D))
    return out


# speedup vs baseline: 15.7699x; 2.1675x over previous
import jax
import jax.numpy as jnp
from jax import lax
from jax.experimental import pallas as pl
from jax.experimental.pallas import tpu as pltpu
from jax.experimental.pallas import tpu_sc as plsc

N = 50000
D = 256
G = 64
NEG_INF = float("-inf")

TCCHUNK = 512
NCH = 69
TCROWS = TCCHUNK * NCH
SCROWS = N - TCROWS

L = 16
CB = D // L
NW = 32
CHUNK = 112
RPW = 448
NPAIRS = RPW // CHUNK // 2
NEXTRA = (SCROWS - NW * RPW) // CHUNK

IDS_PAD = ((N + TCCHUNK - 1) // TCCHUNK) * TCCHUNK


def _tree_reduce(xs, op):
    while len(xs) > 1:
        xs = [op(xs[2 * i], xs[2 * i + 1]) for i in range(len(xs) // 2)] + \
             (xs[-1:] if len(xs) % 2 else [])
    return xs[0]


def _sc_partials_kernel(x_hbm, ids_hbm, psum_hbm, pmax_hbm, pcnt_hbm,
                        x0, x1, i0, i1, sum_v, max_v, cnt_v,
                        sx0, sx1, si0, si1):
    wid = lax.axis_index("s") * 2 + lax.axis_index("c")
    base = TCROWS + wid * RPW + CHUNK * jnp.minimum(wid, NEXTRA)

    zeros16 = jnp.zeros((L,), jnp.float32)
    neg16 = jnp.full((L,), NEG_INF, jnp.float32)
    ones16 = jnp.ones((L,), jnp.float32)

    def init_body(s, _):
        for cb in range(CB):
            sum_v[s, pl.ds(cb * L, L)] = zeros16
            max_v[s, pl.ds(cb * L, L)] = neg16
        cnt_v[s, :] = zeros16
        return 0
    lax.fori_loop(0, G, init_body, 0)

    xb = (x0, x1)
    ib = (i0, i1)
    sxb = (sx0, sx1)
    sib = (si0, si1)

    def start(c, k):
        st = base + c * CHUNK
        pltpu.async_copy(x_hbm.at[pl.ds(st, CHUNK)], xb[k], sxb[k])
        pltpu.async_copy(ids_hbm.at[pl.ds(st, CHUNK)],
                         ib[k].at[pl.ds(0, CHUNK)], sib[k])

    def wait(k):
        pltpu.make_async_copy(x_hbm.at[pl.ds(0, CHUNK)], xb[k], sxb[k]).wait()
        pltpu.make_async_copy(ids_hbm.at[pl.ds(0, CHUNK)],
                              ib[k].at[pl.ds(0, CHUNK)], sib[k]).wait()

    def process(x_v, ids_v, ngroups):
        def group_body(g, _):
            row0 = g * L
            bvec = ids_v[pl.ds(row0, L)]
            b0 = bvec[0]
            uniform = b0 == bvec[L - 1]

            def uniform_path():
                for cb in range(CB):
                    xs = [x_v[row0 + j, pl.ds(cb * L, L)] for j in range(L)]
                    s = _tree_reduce(list(xs), jnp.add)
                    m = _tree_reduce(list(xs), jnp.maximum)
                    plsc.addupdate(sum_v.at[b0, pl.ds(cb * L, L)], s)
                    cur = max_v[b0, pl.ds(cb * L, L)]
                    max_v[b0, pl.ds(cb * L, L)] = jnp.maximum(cur, m)
                plsc.addupdate(cnt_v.at[b0],
                               jnp.full((L,), float(L), jnp.float32))

            def rowwise_path():
                def row_body(j, _):
                    row = row0 + j
                    b = ids_v[pl.ds(row, L)][0]
                    for cb in range(CB):
                        x = x_v[row, pl.ds(cb * L, L)]
                        plsc.addupdate(sum_v.at[b, pl.ds(cb * L, L)], x)
                        cur = max_v[b, pl.ds(cb * L, L)]
                        max_v[b, pl.ds(cb * L, L)] = jnp.maximum(cur, x)
                    plsc.addupdate(cnt_v.at[b], ones16)
                    return 0
                lax.fori_loop(0, L, row_body, 0)

            lax.cond(uniform, uniform_path, rowwise_path)
            return 0
        lax.fori_loop(0, ngroups, group_body, 0)

    start(0, 0)

    def pair_body(p, _):
        c0 = 2 * p
        start(c0 + 1, 1)
        wait(0)
        process(x0, i0, CHUNK // L)

        @pl.when(p + 1 < NPAIRS)
        def _():
            start(c0 + 2, 0)
        wait(1)
        process(x1, i1, CHUNK // L)
        return 0
    lax.fori_loop(0, NPAIRS, pair_body, 0)

    @pl.when(wid < NEXTRA)
    def _():
        st = base + NPAIRS * 2 * CHUNK
        pltpu.sync_copy(x_hbm.at[pl.ds(st, CHUNK)], x0)
        pltpu.sync_copy(ids_hbm.at[pl.ds(st, CHUNK)],
                        i0.at[pl.ds(0, CHUNK)])
        process(x0, i0, CHUNK // L)

    pltpu.sync_copy(sum_v, psum_hbm.at[wid])
    pltpu.sync_copy(max_v, pmax_hbm.at[wid])
    pltpu.sync_copy(cnt_v, pcnt_hbm.at[wid])


def _tc_jagged_kernel(cmin_ref, cmax_ref, ids_ref, x_ref,
                      sum_ref, max_ref, cnt_ref,
                      cur_sum, cur_max, cur_cnt, cur_seg):
    i = pl.program_id(0)
    seg = lax.broadcasted_iota(jnp.int32, (G, 1), 0)

    @pl.when(i == 0)
    def _():
        sum_ref[...] = jnp.zeros_like(sum_ref)
        max_ref[...] = jnp.full_like(max_ref, NEG_INF)
        cnt_ref[...] = jnp.zeros_like(cnt_ref)
        cur_sum[...] = jnp.zeros_like(cur_sum)
        cur_max[...] = jnp.full_like(cur_max, NEG_INF)
        cur_cnt[...] = jnp.zeros_like(cur_cnt)
        cur_seg[0] = cmin_ref[0]

    def flush():
        hit = seg == cur_seg[0]
        oh = hit.astype(jnp.float32)
        sum_ref[...] += oh * cur_sum[...]
        mb = jnp.where(hit, jnp.broadcast_to(cur_max[...], (G, D)), NEG_INF)
        max_ref[...] = jnp.maximum(max_ref[...], mb)
        cnt_ref[...] += oh * cur_cnt[...]

    def reset(new_seg):
        cur_sum[...] = jnp.zeros_like(cur_sum)
        cur_max[...] = jnp.full_like(cur_max, NEG_INF)
        cur_cnt[...] = jnp.zeros_like(cur_cnt)
        cur_seg[0] = new_seg

    lo = cmin_ref[i]
    hi = cmax_ref[i]

    @pl.when(lo == hi)
    def _():
        @pl.when(lo != cur_seg[0])
        def _():
            flush()
            reset(lo)
        x = x_ref[...]
        cur_sum[...] += jnp.sum(x, axis=0, keepdims=True)
        cur_max[...] = jnp.maximum(cur_max[...],
                                   jnp.max(x, axis=0, keepdims=True))
        cur_cnt[...] += float(TCCHUNK)

    @pl.when(lo != hi)
    def _():
        ids_col = ids_ref[...]
        x = x_ref[...]

        def seg_body(s, _):
            @pl.when(s != cur_seg[0])
            def _():
                flush()
                reset(s)
            rm = ids_col == s
            xs = jnp.where(rm, x, 0.0)
            xm = jnp.where(rm, x, NEG_INF)
            cur_sum[...] += jnp.sum(xs, axis=0, keepdims=True)
            cur_max[...] = jnp.maximum(cur_max[...],
                                       jnp.max(xm, axis=0, keepdims=True))
            nrows = jnp.sum(rm.astype(jnp.float32), axis=0, keepdims=True)
            cur_cnt[...] += jnp.broadcast_to(nrows, cur_cnt.shape)
            return 0
        lax.fori_loop(lo, hi + 1, seg_body, 0)

    @pl.when(i == NCH - 1)
    def _():
        flush()


def _combine_kernel(tsum_ref, tmax_ref, tcnt_ref,
                    psum_ref, pmax_ref, pcnt_ref, w_ref, b_ref, out_ref):
    sums = tsum_ref[...] + jnp.sum(psum_ref[...], axis=0)
    maxs = jnp.maximum(tmax_ref[...], jnp.max(pmax_ref[...], axis=0))
    cnts = tcnt_ref[:, 0:1] + jnp.sum(pcnt_ref[...], axis=0)[:, 0:1]
    mean = sums / jnp.maximum(cnts, 1.0)
    maxs = jnp.where(maxs == NEG_INF, 0.0, maxs)
    combined = jnp.concatenate([mean, maxs], axis=1)
    proj = lax.dot_general(combined, w_ref[...],
                           (((1,), (1,)), ((), ())),
                           preferred_element_type=jnp.float32)
    out_ref[...] = proj + b_ref[...]


def kernel(node_embeddings, batch, W, b):
    batch = batch.astype(jnp.int32)
    ids_pad = jnp.pad(batch, (0, IDS_PAD - N))

    sc = pl.kernel(
        _sc_partials_kernel,
        mesh=plsc.VectorSubcoreMesh(core_axis_name="c", subcore_axis_name="s"),
        out_type=[
            jax.ShapeDtypeStruct((NW, G, D), jnp.float32),
            jax.ShapeDtypeStruct((NW, G, D), jnp.float32),
            jax.ShapeDtypeStruct((NW, G, L), jnp.float32),
        ],
        scratch_types=[
            pltpu.VMEM((CHUNK, D), jnp.float32),
            pltpu.VMEM((CHUNK, D), jnp.float32),
            pltpu.VMEM((CHUNK + L,), jnp.int32),
            pltpu.VMEM((CHUNK + L,), jnp.int32),
            pltpu.VMEM((G, D), jnp.float32),
            pltpu.VMEM((G, D), jnp.float32),
            pltpu.VMEM((G, L), jnp.float32),
            pltpu.SemaphoreType.DMA,
            pltpu.SemaphoreType.DMA,
            pltpu.SemaphoreType.DMA,
            pltpu.SemaphoreType.DMA,
        ],
    )
    psum, pmax, pcnt = sc(node_embeddings, batch)

    ids_mat = ids_pad.reshape(-1, TCCHUNK)
    cmin = jnp.min(ids_mat[:NCH], axis=1)
    cmax = jnp.max(ids_mat[:NCH], axis=1)
    ids_col = ids_pad.reshape(-1, 1)

    tsum, tmax, tcnt = pl.pallas_call(
        _tc_jagged_kernel,
        grid_spec=pltpu.PrefetchScalarGridSpec(
            num_scalar_prefetch=2,
            grid=(NCH,),
            in_specs=[
                pl.BlockSpec((TCCHUNK, 1), lambda i, *_: (i, 0)),
                pl.BlockSpec((TCCHUNK, D), lambda i, *_: (i, 0)),
            ],
            out_specs=[
                pl.BlockSpec((G, D), lambda i, *_: (0, 0)),
                pl.BlockSpec((G, D), lambda i, *_: (0, 0)),
                pl.BlockSpec((G, 128), lambda i, *_: (0, 0)),
            ],
            scratch_shapes=[
                pltpu.VMEM((1, D), jnp.float32),
                pltpu.VMEM((1, D), jnp.float32),
                pltpu.VMEM((1, 128), jnp.float32),
                pltpu.SMEM((1,), jnp.int32),
            ],
        ),
        out_shape=[
            jax.ShapeDtypeStruct((G, D), jnp.float32),
            jax.ShapeDtypeStruct((G, D), jnp.float32),
            jax.ShapeDtypeStruct((G, 128), jnp.float32),
        ],
        compiler_params=pltpu.CompilerParams(
            dimension_semantics=("arbitrary",)),
    )(cmin, cmax, ids_col, node_embeddings)

    out = pl.pallas_call(
        _combine_kernel,
        out_shape=jax.ShapeDtypeStruct((G, D), jnp.float32),
    )(tsum, tmax, tcnt, psum, pmax, pcnt, W, b.reshape(1, D))
    return out


# speedup vs baseline: 19.4773x; 1.2351x over previous
import jax
import jax.numpy as jnp
from jax import lax
from jax.experimental import pallas as pl
from jax.experimental.pallas import tpu as pltpu
from jax.experimental.pallas import tpu_sc as plsc

N = 50000
D = 256
G = 64
NEG_INF = float("-inf")

TCCHUNK = 2048
NCH = 17
TCROWS = TCCHUNK * NCH
SCROWS = N - TCROWS

L = 16
CB = D // L
NW = 32
CHUNK = 112
RPW = 448
NPAIRS = RPW // CHUNK // 2
NEXTRA = 7
MINI = SCROWS - NW * RPW - NEXTRA * CHUNK

IDS_PAD = ((N + TCCHUNK - 1) // TCCHUNK) * TCCHUNK


def _tree_reduce(xs, op):
    while len(xs) > 1:
        xs = [op(xs[2 * i], xs[2 * i + 1]) for i in range(len(xs) // 2)] + \
             (xs[-1:] if len(xs) % 2 else [])
    return xs[0]


def _sc_partials_kernel(x_hbm, ids_hbm, psum_hbm, pmax_hbm, pcnt_hbm,
                        x0, x1, i0, i1, sum_v, max_v, cnt_v,
                        sx0, sx1, si0, si1):
    wid = lax.axis_index("s") * 2 + lax.axis_index("c")
    base = (TCROWS + wid * RPW + CHUNK * jnp.minimum(wid, NEXTRA)
            + jnp.where(wid > NEXTRA, MINI, 0))

    zeros16 = jnp.zeros((L,), jnp.float32)
    neg16 = jnp.full((L,), NEG_INF, jnp.float32)
    ones16 = jnp.ones((L,), jnp.float32)

    def init_body(s, _):
        for cb in range(CB):
            sum_v[s, pl.ds(cb * L, L)] = zeros16
            max_v[s, pl.ds(cb * L, L)] = neg16
        cnt_v[s, :] = zeros16
        return 0
    lax.fori_loop(0, G, init_body, 0)

    xb = (x0, x1)
    ib = (i0, i1)
    sxb = (sx0, sx1)
    sib = (si0, si1)

    def start(c, k):
        st = base + c * CHUNK
        pltpu.async_copy(x_hbm.at[pl.ds(st, CHUNK)], xb[k], sxb[k])
        pltpu.async_copy(ids_hbm.at[pl.ds(st, CHUNK)],
                         ib[k].at[pl.ds(0, CHUNK)], sib[k])

    def wait(k):
        pltpu.make_async_copy(x_hbm.at[pl.ds(0, CHUNK)], xb[k], sxb[k]).wait()
        pltpu.make_async_copy(ids_hbm.at[pl.ds(0, CHUNK)],
                              ib[k].at[pl.ds(0, CHUNK)], sib[k]).wait()

    def process(x_v, ids_v, ngroups):
        def group_body(g, _):
            row0 = g * L
            bvec = ids_v[pl.ds(row0, L)]
            b0 = bvec[0]
            uniform = b0 == bvec[L - 1]

            def uniform_path():
                for cb in range(CB):
                    xs = [x_v[row0 + j, pl.ds(cb * L, L)] for j in range(L)]
                    s = _tree_reduce(list(xs), jnp.add)
                    m = _tree_reduce(list(xs), jnp.maximum)
                    plsc.addupdate(sum_v.at[b0, pl.ds(cb * L, L)], s)
                    cur = max_v[b0, pl.ds(cb * L, L)]
                    max_v[b0, pl.ds(cb * L, L)] = jnp.maximum(cur, m)
                plsc.addupdate(cnt_v.at[b0],
                               jnp.full((L,), float(L), jnp.float32))

            def rowwise_path():
                def row_body(j, _):
                    row = row0 + j
                    b = ids_v[pl.ds(row, L)][0]
                    for cb in range(CB):
                        x = x_v[row, pl.ds(cb * L, L)]
                        plsc.addupdate(sum_v.at[b, pl.ds(cb * L, L)], x)
                        cur = max_v[b, pl.ds(cb * L, L)]
                        max_v[b, pl.ds(cb * L, L)] = jnp.maximum(cur, x)
                    plsc.addupdate(cnt_v.at[b], ones16)
                    return 0
                lax.fori_loop(0, L, row_body, 0)

            lax.cond(uniform, uniform_path, rowwise_path)
            return 0
        lax.fori_loop(0, ngroups, group_body, 0)

    start(0, 0)

    def pair_body(p, _):
        c0 = 2 * p
        start(c0 + 1, 1)
        wait(0)
        process(x0, i0, CHUNK // L)

        @pl.when(p + 1 < NPAIRS)
        def _():
            start(c0 + 2, 0)
        wait(1)
        process(x1, i1, CHUNK // L)
        return 0
    lax.fori_loop(0, NPAIRS, pair_body, 0)

    @pl.when(wid < NEXTRA)
    def _():
        st = base + NPAIRS * 2 * CHUNK
        pltpu.sync_copy(x_hbm.at[pl.ds(st, CHUNK)], x0)
        pltpu.sync_copy(ids_hbm.at[pl.ds(st, CHUNK)],
                        i0.at[pl.ds(0, CHUNK)])
        process(x0, i0, CHUNK // L)

    @pl.when(wid == NEXTRA)
    def _():
        st = base + NPAIRS * 2 * CHUNK
        pltpu.sync_copy(x_hbm.at[pl.ds(st, MINI)], x0.at[pl.ds(0, MINI)])
        pltpu.sync_copy(ids_hbm.at[pl.ds(st, MINI)],
                        i0.at[pl.ds(0, MINI)])
        process(x0, i0, MINI // L)

    pltpu.sync_copy(sum_v, psum_hbm.at[wid])
    pltpu.sync_copy(max_v, pmax_hbm.at[wid])
    pltpu.sync_copy(cnt_v, pcnt_hbm.at[wid])


def _tc_jagged_kernel(cmin_ref, cmax_ref, ids_ref, x_ref,
                      sum_ref, max_ref, cnt_ref):
    i = pl.program_id(0)
    seg = lax.broadcasted_iota(jnp.int32, (G, 1), 0)

    @pl.when(i == 0)
    def _():
        sum_ref[...] = jnp.zeros_like(sum_ref)
        max_ref[...] = jnp.full_like(max_ref, NEG_INF)
        cnt_ref[...] = jnp.zeros_like(cnt_ref)

    lo = cmin_ref[i]
    hi = cmax_ref[i]
    ids_col = ids_ref[...]
    x = x_ref[...]

    def seg_body(sv, _):
        rm = ids_col == sv
        ssum = jnp.sum(jnp.where(rm, x, 0.0), axis=0, keepdims=True)
        smax = jnp.max(jnp.where(rm, x, NEG_INF), axis=0, keepdims=True)
        nrows = jnp.sum(rm.astype(jnp.float32), axis=0, keepdims=True)
        hit = seg == sv
        oh = hit.astype(jnp.float32)
        sum_ref[...] += oh * ssum
        max_ref[...] = jnp.maximum(
            max_ref[...],
            jnp.where(hit, jnp.broadcast_to(smax, (G, D)), NEG_INF))
        cnt_ref[...] += oh * jnp.broadcast_to(nrows, (G, 128))
        return 0
    lax.fori_loop(lo, hi + 1, seg_body, 0)


def _combine_kernel(tsum_ref, tmax_ref, tcnt_ref,
                    psum_ref, pmax_ref, pcnt_ref, w_ref, b_ref, out_ref):
    sums = tsum_ref[...] + jnp.sum(psum_ref[...], axis=0)
    maxs = jnp.maximum(tmax_ref[...], jnp.max(pmax_ref[...], axis=0))
    cnts = tcnt_ref[:, 0:1] + jnp.sum(pcnt_ref[...], axis=0)[:, 0:1]
    mean = sums / jnp.maximum(cnts, 1.0)
    maxs = jnp.where(maxs == NEG_INF, 0.0, maxs)
    combined = jnp.concatenate([mean, maxs], axis=1)
    proj = lax.dot_general(combined, w_ref[...],
                           (((1,), (1,)), ((), ())),
                           preferred_element_type=jnp.float32)
    out_ref[...] = proj + b_ref[...]


def kernel(node_embeddings, batch, W, b):
    batch = batch.astype(jnp.int32)
    ids_pad = jnp.pad(batch, (0, IDS_PAD - N))

    sc = pl.kernel(
        _sc_partials_kernel,
        mesh=plsc.VectorSubcoreMesh(core_axis_name="c", subcore_axis_name="s"),
        out_type=[
            jax.ShapeDtypeStruct((NW, G, D), jnp.float32),
            jax.ShapeDtypeStruct((NW, G, D), jnp.float32),
            jax.ShapeDtypeStruct((NW, G, L), jnp.float32),
        ],
        scratch_types=[
            pltpu.VMEM((CHUNK, D), jnp.float32),
            pltpu.VMEM((CHUNK, D), jnp.float32),
            pltpu.VMEM((CHUNK + L,), jnp.int32),
            pltpu.VMEM((CHUNK + L,), jnp.int32),
            pltpu.VMEM((G, D), jnp.float32),
            pltpu.VMEM((G, D), jnp.float32),
            pltpu.VMEM((G, L), jnp.float32),
            pltpu.SemaphoreType.DMA,
            pltpu.SemaphoreType.DMA,
            pltpu.SemaphoreType.DMA,
            pltpu.SemaphoreType.DMA,
        ],
    )
    psum, pmax, pcnt = sc(node_embeddings, batch)

    ids_mat = ids_pad.reshape(-1, TCCHUNK)
    cmin = jnp.min(ids_mat[:NCH], axis=1)
    cmax = jnp.max(ids_mat[:NCH], axis=1)
    ids_col = ids_pad.reshape(-1, 1)

    tsum, tmax, tcnt = pl.pallas_call(
        _tc_jagged_kernel,
        grid_spec=pltpu.PrefetchScalarGridSpec(
            num_scalar_prefetch=2,
            grid=(NCH,),
            in_specs=[
                pl.BlockSpec((TCCHUNK, 1), lambda i, *_: (i, 0)),
                pl.BlockSpec((TCCHUNK, D), lambda i, *_: (i, 0)),
            ],
            out_specs=[
                pl.BlockSpec((G, D), lambda i, *_: (0, 0)),
                pl.BlockSpec((G, D), lambda i, *_: (0, 0)),
                pl.BlockSpec((G, 128), lambda i, *_: (0, 0)),
            ],
        ),
        out_shape=[
            jax.ShapeDtypeStruct((G, D), jnp.float32),
            jax.ShapeDtypeStruct((G, D), jnp.float32),
            jax.ShapeDtypeStruct((G, 128), jnp.float32),
        ],
        compiler_params=pltpu.CompilerParams(
            dimension_semantics=("arbitrary",)),
    )(cmin, cmax, ids_col, node_embeddings)

    out = pl.pallas_call(
        _combine_kernel,
        out_shape=jax.ShapeDtypeStruct((G, D), jnp.float32),
    )(tsum, tmax, tcnt, psum, pmax, pcnt, W, b.reshape(1, D))
    return out
